# Initial kernel scaffold; baseline (speedup 1.0000x reference)
import jax
import jax.numpy as jnp
from jax.experimental import pallas as pl


def kernel(output, mask, ind, target):
    raise NotImplementedError("write your pallas kernel here")


# speedup vs baseline: 2.6528x; 2.6528x over previous
import functools

import jax
import jax.numpy as jnp
from jax import lax
from jax.experimental import pallas as pl
from jax.experimental.pallas import tpu as pltpu
from jax.experimental.pallas import tpu_sc as plsc

_B, _C, _H, _W, _K = 64, 2, 512, 512, 500
_HW = _H * _W
_NW = 32
_BPW = _B // _NW
_PAIRS = _BPW * _K
_ELEMS = _PAIRS * _C
_NCHUNK = _ELEMS // 16
_GCHUNK = 128
_NDMA = -(-_ELEMS // _GCHUNK)
_EPAD = _NDMA * _GCHUNK


def _sc_partials(out_flat, ind2, mask2, tgt3):
  mesh = plsc.VectorSubcoreMesh(core_axis_name="c", subcore_axis_name="s")

  @functools.partial(
      pl.kernel,
      out_type=jax.ShapeDtypeStruct((2, _NW, 16), jnp.float32),
      mesh=mesh,
      compiler_params=pltpu.CompilerParams(needs_layout_passes=False),
      scratch_types=[
          pltpu.VMEM((_PAIRS,), jnp.int32),
          pltpu.VMEM((_PAIRS,), jnp.int32),
          pltpu.VMEM((_EPAD,), jnp.int32),
          pltpu.VMEM((_EPAD,), jnp.float32),
          pltpu.VMEM((_NCHUNK, 16), jnp.float32),
          pltpu.VMEM((16,), jnp.float32),
          pltpu.SemaphoreType.DMA,
      ],
  )
  def k(out_hbm, ind_hbm, mask_hbm, tgt_hbm, part_hbm,
        ind_v, mask_v, gidx_v, pred_v, tgt_v, stage_v, sem):
    wid = lax.axis_index("c") * 16 + lax.axis_index("s")
    pltpu.sync_copy(ind_hbm.at[wid], ind_v)
    pltpu.sync_copy(mask_hbm.at[wid], mask_v)
    pltpu.sync_copy(tgt_hbm.at[wid], tgt_v)

    iota = lax.iota(jnp.int32, 16)
    b0 = wid * _BPW

    def build(i, _):
      j = i * 16 + lax.iota(jnp.int32, 16)
      p = j >> 1
      ch = j & 1
      bl = (j * 33555) >> 25
      indg = plsc.load_gather(ind_v, [p])
      gidx_v[pl.ds(i * 16, 16)] = (b0 + bl) * (_C * _HW) + ch * _HW + indg
      return 0

    lax.fori_loop(0, _NCHUNK, build, 0)
    for i in range(_NCHUNK, _EPAD // 16):
      gidx_v[pl.ds(i * 16, 16)] = jnp.zeros((16,), jnp.int32)

    copies = [
        pltpu.async_copy(
            out_hbm.at[gidx_v.at[pl.ds(t * _GCHUNK, _GCHUNK)]],
            pred_v.at[pl.ds(t * _GCHUNK, _GCHUNK)], sem)
        for t in range(_NDMA)
    ]
    for c in copies:
      c.wait()

    def accum(i, carry):
      a, m = carry
      j = i * 16 + iota
      mg = plsc.load_gather(mask_v, [j >> 1]).astype(jnp.float32)
      a = a + jnp.abs(pred_v[pl.ds(i * 16, 16)] - tgt_v[i, :]) * mg
      return (a, m + mg)

    zero = jnp.zeros((16,), jnp.float32)
    a, m = lax.fori_loop(0, _NCHUNK, accum, (zero, zero))
    stage_v[...] = a
    pltpu.sync_copy(stage_v, part_hbm.at[0, wid])
    stage_v[...] = m
    pltpu.sync_copy(stage_v, part_hbm.at[1, wid])

  return k(out_flat, ind2, mask2, tgt3)


def _finalize(parts):
  def body(p_ref, o_ref):
    p = p_ref[...]
    o_ref[0, 0] = jnp.sum(p[0, :]) / (jnp.sum(p[1, :]) + 0.0001)

  return pl.pallas_call(
      body,
      out_shape=jax.ShapeDtypeStruct((1, 1), jnp.float32),
      out_specs=pl.BlockSpec(memory_space=pltpu.SMEM),
  )(parts)


def kernel(output, mask, ind, target):
  out_flat = output.reshape(-1)
  ind2 = ind.reshape(_NW, _PAIRS)
  mask2 = mask.reshape(_NW, _PAIRS)
  tgt3 = target.reshape(_NW, _NCHUNK, 16)
  parts = _sc_partials(out_flat, ind2, mask2, tgt3)
  return _finalize(parts.reshape(2, _NW * 16))[0, 0]


# speedup vs baseline: 3.3741x; 1.2719x over previous
import functools

import jax
import jax.numpy as jnp
from jax import lax
from jax.experimental import pallas as pl
from jax.experimental.pallas import tpu as pltpu
from jax.experimental.pallas import tpu_sc as plsc

_B, _C, _H, _W, _K = 64, 2, 512, 512, 500
_HW = _H * _W
_NW = 32
_BPW = _B // _NW
_PAIRS = _BPW * _K
_ELEMS = _PAIRS * _C
_NCHUNK = _ELEMS // 16
_ROWS = _BPW * _C * _H
_PROWS = 64
_NPASS = _ROWS // _PROWS


def _sc_partials(out4, ind2, mask2, tgt3):
  mesh = plsc.VectorSubcoreMesh(core_axis_name="c", subcore_axis_name="s")

  @functools.partial(
      pl.kernel,
      out_type=jax.ShapeDtypeStruct((2, _NW, 16), jnp.float32),
      mesh=mesh,
      compiler_params=pltpu.CompilerParams(needs_layout_passes=False),
      scratch_types=[
          pltpu.VMEM((_PAIRS,), jnp.int32),
          pltpu.VMEM((_PAIRS,), jnp.int32),
          pltpu.VMEM((_ELEMS,), jnp.int32),
          pltpu.VMEM((_ELEMS,), jnp.int32),
          pltpu.VMEM((_ELEMS,), jnp.float32),
          pltpu.VMEM((_NCHUNK, 16), jnp.float32),
          pltpu.VMEM((_PROWS, _W), jnp.float32),
          pltpu.VMEM((_PROWS, _W), jnp.float32),
          pltpu.VMEM((16,), jnp.float32),
          pltpu.SemaphoreType.DMA,
          pltpu.SemaphoreType.DMA,
      ],
  )
  def k(out4_hbm, ind_hbm, mask_hbm, tgt_hbm, part_hbm,
        ind_v, mask_v, lr_v, col_v, mgx_v, tgt_v, buf0, buf1, stage_v,
        sem0, sem1):
    out2 = out4_hbm.reshape(_B * _C * _H, _W)
    wid = lax.axis_index("c") * 16 + lax.axis_index("s")
    pltpu.sync_copy(ind_hbm.at[wid], ind_v)
    pltpu.sync_copy(mask_hbm.at[wid], mask_v)
    pltpu.sync_copy(tgt_hbm.at[wid], tgt_v)

    iota = lax.iota(jnp.int32, 16)

    def build(i, macc):
      j = i * 16 + iota
      p = j >> 1
      ch = j & 1
      bl = (j * 33555) >> 25
      hw = plsc.load_gather(ind_v, [p])
      mgx = plsc.load_gather(mask_v, [p]).astype(jnp.float32)
      lr_v[pl.ds(i * 16, 16)] = ((bl * _C + ch) << 9) + (hw >> 9)
      col_v[pl.ds(i * 16, 16)] = hw & (_W - 1)
      mgx_v[pl.ds(i * 16, 16)] = mgx
      return macc + mgx

    zero = jnp.zeros((16,), jnp.float32)
    macc = lax.fori_loop(0, _NCHUNK, build, zero)

    base0 = pl.multiple_of(wid * _ROWS, _PROWS)
    bufs = (buf0, buf1)
    sems = (sem0, sem1)
    cps = [pltpu.async_copy(out2.at[pl.ds(base0, _PROWS)], buf0, sem0),
           None]

    def sweep(pass_id, buf):
      def body(i, a):
        lr = lr_v[pl.ds(i * 16, 16)]
        col = col_v[pl.ds(i * 16, 16)]
        meff = jnp.where((lr >> 6) == pass_id, mgx_v[pl.ds(i * 16, 16)], 0.0)
        val = plsc.load_gather(buf, [lr & (_PROWS - 1), col])
        return a + jnp.abs(val - tgt_v[i, :]) * meff
      return body

    acc = zero
    for pss in range(_NPASS):
      cur = pss & 1
      cps[cur].wait()
      if pss + 1 < _NPASS:
        nxt = (pss + 1) & 1
        base = pl.multiple_of(wid * _ROWS + (pss + 1) * _PROWS, _PROWS)
        cps[nxt] = pltpu.async_copy(out2.at[pl.ds(base, _PROWS)],
                                    bufs[nxt], sems[nxt])
      acc = lax.fori_loop(0, _NCHUNK, sweep(pss, bufs[cur]), acc)

    stage_v[...] = acc
    pltpu.sync_copy(stage_v, part_hbm.at[0, wid])
    stage_v[...] = macc
    pltpu.sync_copy(stage_v, part_hbm.at[1, wid])

  return k(out4, ind2, mask2, tgt3)


def _finalize(parts):
  def body(p_ref, o_ref):
    p = p_ref[...]
    o_ref[0, 0] = jnp.sum(p[0, :]) / (jnp.sum(p[1, :]) + 0.0001)

  return pl.pallas_call(
      body,
      out_shape=jax.ShapeDtypeStruct((1, 1), jnp.float32),
      out_specs=pl.BlockSpec(memory_space=pltpu.SMEM),
  )(parts)


def kernel(output, mask, ind, target):
  ind2 = ind.reshape(_NW, _PAIRS)
  mask2 = mask.reshape(_NW, _PAIRS)
  tgt3 = target.reshape(_NW, _NCHUNK, 16)
  parts = _sc_partials(output, ind2, mask2, tgt3)
  return _finalize(parts.reshape(2, _NW * 16))[0, 0]
